# Initial kernel scaffold; baseline (speedup 1.0000x reference)
import jax
import jax.numpy as jnp
from jax.experimental import pallas as pl


def kernel(z, v_in, g_in, b_in, v_out, g_out, b_out, codebook):
    raise NotImplementedError("write your pallas kernel here")


# speedup vs baseline: 1.0271x; 1.0271x over previous
import jax
import jax.numpy as jnp
from jax.experimental import pallas as pl
from jax.experimental.pallas import tpu as pltpu
from jax.experimental.pallas import tpu_sc as plsc

_B, _IN, _T = 4, 1024, 4096
_D = 8
_K = 8192
_TBLK = 128
_CPAD = 128


def _half_argmax(nd_half, offset):
    m = jnp.max(nd_half, axis=1, keepdims=True)
    lane = jax.lax.broadcasted_iota(jnp.int32, nd_half.shape, 1)
    first = jnp.min(jnp.where(nd_half == m, lane, _K), axis=1) + offset
    return m[:, 0], first


def _encode_body(z_ref, w_in_ref, b_in_ref, cbn_t_ref, cb_sq_ref, ze_ref, idx_ref):
    z_blk = z_ref[0]
    z_e = jax.lax.dot_general(
        w_in_ref[...], z_blk, (((1,), (0,)), ((), ())),
        preferred_element_type=jnp.float32)
    z_e = z_e + b_in_ref[...]
    ze_ref[0] = z_e
    enc = z_e.T
    norm = jnp.sqrt(jnp.sum(enc * enc, axis=1, keepdims=True))
    enc_n = enc / jnp.maximum(norm, 1e-12)
    sims = jax.lax.dot_general(
        enc_n, cbn_t_ref[...], (((1,), (0,)), ((), ())),
        preferred_element_type=jnp.float32)
    e_sq = jnp.sum(enc_n * enc_n, axis=1, keepdims=True)
    neg_dist = -((e_sq - 2.0 * sims) + cb_sq_ref[...])
    m1, i1 = _half_argmax(neg_dist[:, : _K // 2], 0)
    m2, i2 = _half_argmax(neg_dist[:, _K // 2:], _K // 2)
    m1b = m1.astype(jnp.bfloat16).astype(jnp.float32)
    first = jnp.where(m2 > m1b, i2, i1)
    idx_ref[0, 0] = first.astype(jnp.int32)


def _decode_body(zq_ref, w_out_ref, b_out_ref, out_ref):
    zq = zq_ref[0][:, :_D]
    out = jax.lax.dot_general(
        w_out_ref[...], zq, (((1,), (1,)), ((), ())),
        preferred_element_type=jnp.float32)
    out_ref[0] = out + b_out_ref[...]


def _sc_gather(cb_pad, idx_flat):
    n_idx = idx_flat.shape[1]
    window = 128

    @pl.kernel(
        out_type=jax.ShapeDtypeStruct((n_idx, _CPAD), jnp.float32),
        mesh=plsc.VectorSubcoreMesh(core_axis_name="core",
                                    subcore_axis_name="subcore"))
    def gather_kernel(cb_hbm, i_hbm, o_hbm):
        def body(i_vmem, o_vmem):
            pltpu.sync_copy(cb_hbm.at[i_vmem.at[0]], o_vmem)

        pltpu.emit_pipeline(
            body,
            grid=(n_idx // window,),
            in_specs=[pl.BlockSpec((1, window), index_map=lambda i: (0, i))],
            out_specs=[pl.BlockSpec((window, _CPAD), index_map=lambda i: (i, 0))],
            core_axis_name=("core", "subcore"),
            dimension_semantics=(pltpu.PARALLEL,),
        )(i_hbm, o_hbm)

    return gather_kernel(cb_pad, idx_flat)


def kernel(z, v_in, g_in, b_in, v_out, g_out, b_out, codebook):
    w_in = g_in[:, None] * v_in / jnp.sqrt(jnp.sum(v_in * v_in, axis=1, keepdims=True))
    w_out = g_out[:, None] * v_out / jnp.sqrt(jnp.sum(v_out * v_out, axis=1, keepdims=True))
    cb_n = codebook / jnp.maximum(jnp.linalg.norm(codebook, axis=1, keepdims=True), 1e-12)
    cb_sq = jnp.sum(cb_n ** 2, axis=1)[None, :]

    grid = (_B, _T // _TBLK)
    z_e, idx3 = pl.pallas_call(
        _encode_body,
        grid=grid,
        in_specs=[
            pl.BlockSpec((1, _IN, _TBLK), lambda b, t: (b, 0, t)),
            pl.BlockSpec((_D, _IN), lambda b, t: (0, 0)),
            pl.BlockSpec((_D, 1), lambda b, t: (0, 0)),
            pl.BlockSpec((_D, _K), lambda b, t: (0, 0)),
            pl.BlockSpec((1, _K), lambda b, t: (0, 0)),
        ],
        out_specs=[
            pl.BlockSpec((1, _D, _TBLK), lambda b, t: (b, 0, t)),
            pl.BlockSpec((1, 1, _TBLK), lambda b, t: (b, 0, t)),
        ],
        out_shape=[
            jax.ShapeDtypeStruct((_B, _D, _T), jnp.float32),
            jax.ShapeDtypeStruct((_B, 1, _T), jnp.int32),
        ],
    )(z, w_in, b_in[:, None], cb_n.T, cb_sq)

    indices = idx3.reshape(_B, _T)

    cb_pad = jnp.concatenate(
        [codebook, jnp.zeros((_K, _CPAD - _D), jnp.float32)], axis=1)
    zq_flat = _sc_gather(cb_pad, idx3.reshape(1, _B * _T))
    zq = zq_flat.reshape(_B, _T, _CPAD)

    z_q_out = pl.pallas_call(
        _decode_body,
        grid=grid,
        in_specs=[
            pl.BlockSpec((1, _TBLK, _CPAD), lambda b, t: (b, t, 0)),
            pl.BlockSpec((_IN, _D), lambda b, t: (0, 0)),
            pl.BlockSpec((_IN, 1), lambda b, t: (0, 0)),
        ],
        out_specs=pl.BlockSpec((1, _IN, _TBLK), lambda b, t: (b, 0, t)),
        out_shape=jax.ShapeDtypeStruct((_B, _IN, _T), jnp.float32),
    )(zq, w_out, b_out[:, None])

    commit_loss = jnp.zeros((_B,), dtype=jnp.float32)
    codebook_loss = jnp.zeros((_B,), dtype=jnp.float32)
    return (z_q_out, commit_loss, codebook_loss, indices, z_e)


# speedup vs baseline: 1.3251x; 1.2901x over previous
import jax
import jax.numpy as jnp
from jax.experimental import pallas as pl
from jax.experimental.pallas import tpu as pltpu
from jax.experimental.pallas import tpu_sc as plsc

_B, _IN, _T = 4, 1024, 4096
_D = 8
_K = 8192
_TBLK = 256
_CPAD = 128


def _half_argmin(d_half, offset):
    m = jnp.min(d_half, axis=1, keepdims=True)
    lane = jax.lax.broadcasted_iota(jnp.int32, d_half.shape, 1)
    first = jnp.min(jnp.where(d_half == m, lane, _K), axis=1) + offset
    return m[:, 0], first


def _encode_body(z_ref, w_in_ref, b_in_ref, cbn_t_ref, cb_sq_ref, ze_ref, idx_ref):
    z_blk = z_ref[0]
    z_e = jax.lax.dot_general(
        w_in_ref[...], z_blk, (((1,), (0,)), ((), ())),
        preferred_element_type=jnp.float32)
    z_e = z_e + b_in_ref[...]
    ze_ref[0] = z_e
    enc = z_e.T
    norm = jnp.sqrt(jnp.sum(enc * enc, axis=1, keepdims=True))
    enc_n = enc / jnp.maximum(norm, 1e-12)
    sims = jax.lax.dot_general(
        enc_n, cbn_t_ref[...], (((1,), (0,)), ((), ())),
        preferred_element_type=jnp.float32)
    e_sq = jnp.sum(enc_n * enc_n, axis=1, keepdims=True)
    dist = (e_sq - 2.0 * sims) + cb_sq_ref[...]
    m1, i1 = _half_argmin(dist[:, : _K // 2], 0)
    m2, i2 = _half_argmin(dist[:, _K // 2:], _K // 2)
    m1b = m1.astype(jnp.bfloat16).astype(jnp.float32)
    first = jnp.where(m2 < m1b, i2, i1)
    idx_ref[0, 0] = first.astype(jnp.int32)


def _decode_body(zq_ref, w_out_ref, b_out_ref, out_ref):
    zq = zq_ref[0][:, :_D]
    out = jax.lax.dot_general(
        w_out_ref[...], zq, (((1,), (1,)), ((), ())),
        preferred_element_type=jnp.float32)
    out_ref[0] = out + b_out_ref[...]


def _sc_gather(cb_pad, idx_flat):
    n_idx = idx_flat.shape[1]
    window = 128

    @pl.kernel(
        out_type=jax.ShapeDtypeStruct((n_idx, _CPAD), jnp.float32),
        mesh=plsc.VectorSubcoreMesh(core_axis_name="core",
                                    subcore_axis_name="subcore"))
    def gather_kernel(cb_hbm, i_hbm, o_hbm):
        def body(i_vmem, o_vmem):
            pltpu.sync_copy(cb_hbm.at[i_vmem.at[0]], o_vmem)

        pltpu.emit_pipeline(
            body,
            grid=(n_idx // window,),
            in_specs=[pl.BlockSpec((1, window), index_map=lambda i: (0, i))],
            out_specs=[pl.BlockSpec((window, _CPAD), index_map=lambda i: (i, 0))],
            core_axis_name=("core", "subcore"),
            dimension_semantics=(pltpu.PARALLEL,),
        )(i_hbm, o_hbm)

    return gather_kernel(cb_pad, idx_flat)


def kernel(z, v_in, g_in, b_in, v_out, g_out, b_out, codebook):
    w_in = g_in[:, None] * v_in / jnp.sqrt(jnp.sum(v_in * v_in, axis=1, keepdims=True))
    w_out = g_out[:, None] * v_out / jnp.sqrt(jnp.sum(v_out * v_out, axis=1, keepdims=True))
    cb_n = codebook / jnp.maximum(jnp.linalg.norm(codebook, axis=1, keepdims=True), 1e-12)
    cb_sq = jnp.sum(cb_n ** 2, axis=1)[None, :]

    grid = (_B, _T // _TBLK)
    z_e, idx3 = pl.pallas_call(
        _encode_body,
        grid=grid,
        in_specs=[
            pl.BlockSpec((1, _IN, _TBLK), lambda b, t: (b, 0, t)),
            pl.BlockSpec((_D, _IN), lambda b, t: (0, 0)),
            pl.BlockSpec((_D, 1), lambda b, t: (0, 0)),
            pl.BlockSpec((_D, _K), lambda b, t: (0, 0)),
            pl.BlockSpec((1, _K), lambda b, t: (0, 0)),
        ],
        out_specs=[
            pl.BlockSpec((1, _D, _TBLK), lambda b, t: (b, 0, t)),
            pl.BlockSpec((1, 1, _TBLK), lambda b, t: (b, 0, t)),
        ],
        out_shape=[
            jax.ShapeDtypeStruct((_B, _D, _T), jnp.float32),
            jax.ShapeDtypeStruct((_B, 1, _T), jnp.int32),
        ],
    )(z, w_in, b_in[:, None], cb_n.T, cb_sq)

    indices = idx3.reshape(_B, _T)

    cb_pad = jnp.concatenate(
        [codebook, jnp.zeros((_K, _CPAD - _D), jnp.float32)], axis=1)
    zq_flat = _sc_gather(cb_pad, idx3.reshape(1, _B * _T))
    zq = zq_flat.reshape(_B, _T, _CPAD)

    z_q_out = pl.pallas_call(
        _decode_body,
        grid=grid,
        in_specs=[
            pl.BlockSpec((1, _TBLK, _CPAD), lambda b, t: (b, t, 0)),
            pl.BlockSpec((_IN, _D), lambda b, t: (0, 0)),
            pl.BlockSpec((_IN, 1), lambda b, t: (0, 0)),
        ],
        out_specs=pl.BlockSpec((1, _IN, _TBLK), lambda b, t: (b, 0, t)),
        out_shape=jax.ShapeDtypeStruct((_B, _IN, _T), jnp.float32),
    )(zq, w_out, b_out[:, None])

    commit_loss = jnp.zeros((_B,), dtype=jnp.float32)
    codebook_loss = jnp.zeros((_B,), dtype=jnp.float32)
    return (z_q_out, commit_loss, codebook_loss, indices, z_e)


# speedup vs baseline: 1.3614x; 1.0274x over previous
import jax
import jax.numpy as jnp
from jax.experimental import pallas as pl
from jax.experimental.pallas import tpu as pltpu
from jax.experimental.pallas import tpu_sc as plsc

_B, _IN, _T = 4, 1024, 4096
_D = 8
_K = 8192
_TBLK = 256
_CPAD = 128


def _half_argmin(d_half, offset):
    m = jnp.min(d_half, axis=1)
    first = jnp.argmin(d_half, axis=1).astype(jnp.int32) + offset
    return m, first


def _encode_body(z_ref, w_in_ref, b_in_ref, cbn_t_ref, cb_sq_ref, ze_ref, idx_ref):
    z_blk = z_ref[0]
    z_e = jax.lax.dot_general(
        w_in_ref[...], z_blk, (((1,), (0,)), ((), ())),
        preferred_element_type=jnp.float32)
    z_e = z_e + b_in_ref[...]
    ze_ref[0] = z_e
    enc = z_e.T
    norm = jnp.sqrt(jnp.sum(enc * enc, axis=1, keepdims=True))
    enc_n = enc / jnp.maximum(norm, 1e-12)
    sims = jax.lax.dot_general(
        enc_n, cbn_t_ref[...], (((1,), (0,)), ((), ())),
        preferred_element_type=jnp.float32)
    e_sq = jnp.sum(enc_n * enc_n, axis=1, keepdims=True)
    dist = (e_sq - 2.0 * sims) + cb_sq_ref[...]
    m1, i1 = _half_argmin(dist[:, : _K // 2], 0)
    m2, i2 = _half_argmin(dist[:, _K // 2:], _K // 2)
    m1b = m1.astype(jnp.bfloat16).astype(jnp.float32)
    first = jnp.where(m2 < m1b, i2, i1)
    idx_ref[0, 0] = first.astype(jnp.int32)


def _decode_body(zq_ref, w_out_ref, b_out_ref, out_ref):
    zq = zq_ref[0][:, :_D]
    out = jax.lax.dot_general(
        w_out_ref[...], zq, (((1,), (1,)), ((), ())),
        preferred_element_type=jnp.float32)
    out_ref[0] = out + b_out_ref[...]


def _sc_gather(cb_pad, idx_flat):
    n_idx = idx_flat.shape[1]
    window = 128

    @pl.kernel(
        out_type=jax.ShapeDtypeStruct((n_idx, _CPAD), jnp.float32),
        mesh=plsc.VectorSubcoreMesh(core_axis_name="core",
                                    subcore_axis_name="subcore"))
    def gather_kernel(cb_hbm, i_hbm, o_hbm):
        def body(i_vmem, o_vmem):
            pltpu.sync_copy(cb_hbm.at[i_vmem.at[0]], o_vmem)

        pltpu.emit_pipeline(
            body,
            grid=(n_idx // window,),
            in_specs=[pl.BlockSpec((1, window), index_map=lambda i: (0, i))],
            out_specs=[pl.BlockSpec((window, _CPAD), index_map=lambda i: (i, 0))],
            core_axis_name=("core", "subcore"),
            dimension_semantics=(pltpu.PARALLEL,),
        )(i_hbm, o_hbm)

    return gather_kernel(cb_pad, idx_flat)


def kernel(z, v_in, g_in, b_in, v_out, g_out, b_out, codebook):
    w_in = g_in[:, None] * v_in / jnp.sqrt(jnp.sum(v_in * v_in, axis=1, keepdims=True))
    w_out = g_out[:, None] * v_out / jnp.sqrt(jnp.sum(v_out * v_out, axis=1, keepdims=True))
    cb_n = codebook / jnp.maximum(jnp.linalg.norm(codebook, axis=1, keepdims=True), 1e-12)
    cb_sq = jnp.sum(cb_n ** 2, axis=1)[None, :]

    grid = (_B, _T // _TBLK)
    z_e, idx3 = pl.pallas_call(
        _encode_body,
        grid=grid,
        in_specs=[
            pl.BlockSpec((1, _IN, _TBLK), lambda b, t: (b, 0, t)),
            pl.BlockSpec((_D, _IN), lambda b, t: (0, 0)),
            pl.BlockSpec((_D, 1), lambda b, t: (0, 0)),
            pl.BlockSpec((_D, _K), lambda b, t: (0, 0)),
            pl.BlockSpec((1, _K), lambda b, t: (0, 0)),
        ],
        out_specs=[
            pl.BlockSpec((1, _D, _TBLK), lambda b, t: (b, 0, t)),
            pl.BlockSpec((1, 1, _TBLK), lambda b, t: (b, 0, t)),
        ],
        out_shape=[
            jax.ShapeDtypeStruct((_B, _D, _T), jnp.float32),
            jax.ShapeDtypeStruct((_B, 1, _T), jnp.int32),
        ],
    )(z, w_in, b_in[:, None], cb_n.T, cb_sq)

    indices = idx3.reshape(_B, _T)

    cb_pad = jnp.concatenate(
        [codebook, jnp.zeros((_K, _CPAD - _D), jnp.float32)], axis=1)
    zq_flat = _sc_gather(cb_pad, idx3.reshape(1, _B * _T))
    zq = zq_flat.reshape(_B, _T, _CPAD)

    z_q_out = pl.pallas_call(
        _decode_body,
        grid=grid,
        in_specs=[
            pl.BlockSpec((1, _TBLK, _CPAD), lambda b, t: (b, t, 0)),
            pl.BlockSpec((_IN, _D), lambda b, t: (0, 0)),
            pl.BlockSpec((_IN, 1), lambda b, t: (0, 0)),
        ],
        out_specs=pl.BlockSpec((1, _IN, _TBLK), lambda b, t: (b, 0, t)),
        out_shape=jax.ShapeDtypeStruct((_B, _IN, _T), jnp.float32),
    )(zq, w_out, b_out[:, None])

    commit_loss = jnp.zeros((_B,), dtype=jnp.float32)
    codebook_loss = jnp.zeros((_B,), dtype=jnp.float32)
    return (z_q_out, commit_loss, codebook_loss, indices, z_e)


# speedup vs baseline: 1.5575x; 1.1441x over previous
import jax
import jax.numpy as jnp
from jax.experimental import pallas as pl
from jax.experimental.pallas import tpu as pltpu
from jax.experimental.pallas import tpu_sc as plsc

_B, _IN, _T = 4, 1024, 4096
_D = 8
_K = 8192
_TBLK = 512
_CPAD = 128


def _half_argmin(d_half, offset):
    m = jnp.min(d_half, axis=1)
    first = jnp.argmin(d_half, axis=1).astype(jnp.int32) + offset
    return m, first


def _encode_body(z_ref, w_in_ref, b_in_ref, cbn_t_ref, cb_sq_ref, ze_ref, idx_ref):
    z_blk = z_ref[0]
    z_e = jax.lax.dot_general(
        w_in_ref[...], z_blk, (((1,), (0,)), ((), ())),
        preferred_element_type=jnp.float32)
    z_e = z_e + b_in_ref[...]
    ze_ref[0] = z_e
    enc = z_e.T
    norm = jnp.sqrt(jnp.sum(enc * enc, axis=1, keepdims=True))
    enc_n = enc / jnp.maximum(norm, 1e-12)
    sims = jax.lax.dot_general(
        enc_n, cbn_t_ref[...], (((1,), (0,)), ((), ())),
        preferred_element_type=jnp.float32)
    e_sq = jnp.sum(enc_n * enc_n, axis=1, keepdims=True)
    dist = (e_sq - 2.0 * sims) + cb_sq_ref[...]
    m1, i1 = _half_argmin(dist[:, : _K // 2], 0)
    m2, i2 = _half_argmin(dist[:, _K // 2:], _K // 2)
    m1b = m1.astype(jnp.bfloat16).astype(jnp.float32)
    first = jnp.where(m2 < m1b, i2, i1)
    idx_ref[0, 0] = first.astype(jnp.int32)


def _decode_body(zq_ref, w_out_ref, b_out_ref, out_ref):
    zq = zq_ref[0][:, :_D]
    out = jax.lax.dot_general(
        w_out_ref[...], zq, (((1,), (1,)), ((), ())),
        preferred_element_type=jnp.float32)
    out_ref[0] = out + b_out_ref[...]


def _sc_gather(cb_pad, idx_flat):
    n_idx = idx_flat.shape[1]
    window = 256

    @pl.kernel(
        out_type=jax.ShapeDtypeStruct((n_idx, _CPAD), jnp.float32),
        mesh=plsc.VectorSubcoreMesh(core_axis_name="core",
                                    subcore_axis_name="subcore"))
    def gather_kernel(cb_hbm, i_hbm, o_hbm):
        def body(i_vmem, o_vmem):
            pltpu.sync_copy(cb_hbm.at[i_vmem.at[0]], o_vmem)

        pltpu.emit_pipeline(
            body,
            grid=(n_idx // window,),
            in_specs=[pl.BlockSpec((1, window), index_map=lambda i: (0, i))],
            out_specs=[pl.BlockSpec((window, _CPAD), index_map=lambda i: (i, 0))],
            core_axis_name=("core", "subcore"),
            dimension_semantics=(pltpu.PARALLEL,),
        )(i_hbm, o_hbm)

    return gather_kernel(cb_pad, idx_flat)


def kernel(z, v_in, g_in, b_in, v_out, g_out, b_out, codebook):
    w_in = g_in[:, None] * v_in / jnp.sqrt(jnp.sum(v_in * v_in, axis=1, keepdims=True))
    w_out = g_out[:, None] * v_out / jnp.sqrt(jnp.sum(v_out * v_out, axis=1, keepdims=True))
    cb_n = codebook / jnp.maximum(jnp.linalg.norm(codebook, axis=1, keepdims=True), 1e-12)
    cb_sq = jnp.sum(cb_n ** 2, axis=1)[None, :]

    grid = (_B, _T // _TBLK)
    z_e, idx3 = pl.pallas_call(
        _encode_body,
        grid=grid,
        in_specs=[
            pl.BlockSpec((1, _IN, _TBLK), lambda b, t: (b, 0, t)),
            pl.BlockSpec((_D, _IN), lambda b, t: (0, 0)),
            pl.BlockSpec((_D, 1), lambda b, t: (0, 0)),
            pl.BlockSpec((_D, _K), lambda b, t: (0, 0)),
            pl.BlockSpec((1, _K), lambda b, t: (0, 0)),
        ],
        out_specs=[
            pl.BlockSpec((1, _D, _TBLK), lambda b, t: (b, 0, t)),
            pl.BlockSpec((1, 1, _TBLK), lambda b, t: (b, 0, t)),
        ],
        out_shape=[
            jax.ShapeDtypeStruct((_B, _D, _T), jnp.float32),
            jax.ShapeDtypeStruct((_B, 1, _T), jnp.int32),
        ],
    )(z, w_in, b_in[:, None], cb_n.T, cb_sq)

    indices = idx3.reshape(_B, _T)

    cb_pad = jnp.concatenate(
        [codebook, jnp.zeros((_K, _CPAD - _D), jnp.float32)], axis=1)
    zq_flat = _sc_gather(cb_pad, idx3.reshape(1, _B * _T))
    zq = zq_flat.reshape(_B, _T, _CPAD)

    z_q_out = pl.pallas_call(
        _decode_body,
        grid=grid,
        in_specs=[
            pl.BlockSpec((1, _TBLK, _CPAD), lambda b, t: (b, t, 0)),
            pl.BlockSpec((_IN, _D), lambda b, t: (0, 0)),
            pl.BlockSpec((_IN, 1), lambda b, t: (0, 0)),
        ],
        out_specs=pl.BlockSpec((1, _IN, _TBLK), lambda b, t: (b, 0, t)),
        out_shape=jax.ShapeDtypeStruct((_B, _IN, _T), jnp.float32),
    )(zq, w_out, b_out[:, None])

    commit_loss = jnp.zeros((_B,), dtype=jnp.float32)
    codebook_loss = jnp.zeros((_B,), dtype=jnp.float32)
    return (z_q_out, commit_loss, codebook_loss, indices, z_e)
